# baseline (speedup 1.0000x reference)
import functools

import jax
import jax.numpy as jnp
from jax import lax
from jax.experimental import pallas as pl
from jax.experimental.pallas import tpu as pltpu
from jax.experimental.pallas import tpu_sc as plsc

NC = 2
NS = 16
NW = NC * NS
GL = 128
NGROUPS = 1600
GPW = NGROUPS // NW


def _embed_lookup(table, idxg):
    _, D = table.shape
    assert D == 64 and idxg.shape == (NGROUPS, GL)
    mesh = plsc.VectorSubcoreMesh(core_axis_name="c", subcore_axis_name="s")

    @functools.partial(
        pl.kernel,
        mesh=mesh,
        compiler_params=pltpu.CompilerParams(use_tc_tiling_on_sc=False,
                                             needs_layout_passes=False),
        out_type=jax.ShapeDtypeStruct((50, 8, 32, 8, 128), jnp.float32),
        scratch_types=[
            pltpu.VMEM((GPW, GL), jnp.int32),
            pltpu.VMEM((GL, D), jnp.float32),
            pltpu.VMEM((GL, D), jnp.float32),
            pltpu.VMEM((8, 8, GL), jnp.float32),
            pltpu.VMEM((8, 8, GL), jnp.float32),
            pltpu.SemaphoreType.DMA,
            pltpu.SemaphoreType.DMA,
            pltpu.SemaphoreType.DMA,
            pltpu.SemaphoreType.DMA,
        ],
    )
    def k(table_hbm, idx_hbm, out_hbm, idx_all, g0, g1, t0, t1,
          gs0, gs1, ws0, ws1):
        gbuf, tbuf = (g0, g1), (t0, t1)
        gsem, wsem = (gs0, gs1), (ws0, ws1)
        wid = lax.axis_index("s") * NC + lax.axis_index("c")
        gbase = wid * GPW
        pltpu.sync_copy(idx_hbm.at[pl.ds(gbase, GPW)], idx_all)

        riota = [lax.iota(jnp.int32, 16) + 16 * l0 for l0 in range(8)]

        def gather_start(j, b):
            pltpu.async_copy(table_hbm.at[idx_all.at[j]], gbuf[b], gsem[b])

        def gather_wait(b):
            pltpu.make_async_copy(table_hbm.at[pl.ds(0, GL)], gbuf[b],
                                  gsem[b]).wait()

        def assemble(b):
            for r in range(8):
                for s in range(8):
                    col = jnp.full((16,), 8 * r + s, jnp.int32)
                    for l0 in range(8):
                        vals = plsc.load_gather(gbuf[b], [riota[l0], col])
                        tbuf[b][r, s, pl.ds(16 * l0, 16)] = vals

        def write_start(g, b):
            h = g // 32
            c = g % 32
            pltpu.async_copy(tbuf[b], out_hbm.at[h, :, c], wsem[b])

        def write_wait(b):
            pltpu.make_async_copy(tbuf[b], out_hbm.at[0, :, 0],
                                  wsem[b]).wait()

        for b in range(2):
            gather_start(b, b)

        def pair(i, carry):
            for b in range(2):
                j = 2 * i + b
                gather_wait(b)

                @pl.when(j >= 2)
                def _():
                    write_wait(b)

                assemble(b)
                write_start(gbase + j, b)

                @pl.when(j <= GPW - 3)
                def _():
                    gather_start(j + 2, b)

            return carry

        lax.fori_loop(0, GPW // 2, pair, 0)
        for b in range(2):
            write_wait(b)

    return k(table, idxg)


def kernel(x, embed_weight):
    batch, hist = x.shape
    _, d = embed_weight.shape
    xt = jnp.swapaxes(x, 0, 1).astype(jnp.int32)
    idxg = xt.reshape(NGROUPS, GL)
    out5 = _embed_lookup(embed_weight, idxg)
    return jnp.transpose(out5, (2, 4, 0, 1, 3)).reshape(batch, hist, d)


# speedup vs baseline: 1.1186x; 1.1186x over previous
import functools

import jax
import jax.numpy as jnp
from jax import lax
from jax.experimental import pallas as pl
from jax.experimental.pallas import tpu as pltpu
from jax.experimental.pallas import tpu_sc as plsc

NC = 2
NS = 16
NW = NC * NS
GL = 128
NGROUPS = 1600
GPW = NGROUPS // NW


def _embed_lookup(table, idxg):
    _, D = table.shape
    assert D == 64 and idxg.shape == (NGROUPS, GL)
    mesh = plsc.VectorSubcoreMesh(core_axis_name="c", subcore_axis_name="s")

    @functools.partial(
        pl.kernel,
        mesh=mesh,
        compiler_params=pltpu.CompilerParams(use_tc_tiling_on_sc=False,
                                             needs_layout_passes=False),
        out_type=jax.ShapeDtypeStruct((50, 8, 32, 8, 128), jnp.float32),
        scratch_types=[
            pltpu.VMEM((GPW, GL), jnp.int32),
            pltpu.VMEM((GL, D), jnp.float32),
            pltpu.VMEM((GL, D), jnp.float32),
            pltpu.VMEM((8, 8, GL), jnp.float32),
            pltpu.VMEM((8, 8, GL), jnp.float32),
            pltpu.SemaphoreType.DMA,
            pltpu.SemaphoreType.DMA,
            pltpu.SemaphoreType.DMA,
            pltpu.SemaphoreType.DMA,
        ],
    )
    def k(table_hbm, idx_hbm, out_hbm, idx_all, g0, g1, t0, t1,
          gs0, gs1, ws0, ws1):
        gbuf, tbuf = (g0, g1), (t0, t1)
        gsem, wsem = (gs0, gs1), (ws0, ws1)
        wid = lax.axis_index("s") * NC + lax.axis_index("c")
        gbase = wid * GPW
        pltpu.sync_copy(idx_hbm.at[pl.ds(gbase, GPW)], idx_all)

        riota = [lax.iota(jnp.int32, 16) + 16 * l0 for l0 in range(8)]

        def gather_start(j, b):
            pltpu.async_copy(table_hbm.at[idx_all.at[j]], gbuf[b], gsem[b])

        def gather_wait(b):
            pltpu.make_async_copy(table_hbm.at[pl.ds(0, GL)], gbuf[b],
                                  gsem[b]).wait()

        def assemble(b):
            for r in range(8):
                for s in range(8):
                    col = jnp.full((16,), 8 * r + s, jnp.int32)
                    vals = [plsc.load_gather(gbuf[b], [riota[l0], col])
                            for l0 in range(8)]
                    for l0 in range(8):
                        tbuf[b][r, s, pl.ds(16 * l0, 16)] = vals[l0]

        def write_start(g, b):
            h = g // 32
            c = g % 32
            pltpu.async_copy(tbuf[b], out_hbm.at[h, :, c], wsem[b])

        def write_wait(b):
            pltpu.make_async_copy(tbuf[b], out_hbm.at[0, :, 0],
                                  wsem[b]).wait()

        for b in range(2):
            gather_start(b, b)

        def pair(i, carry):
            for b in range(2):
                j = 2 * i + b
                gather_wait(b)

                @pl.when(j >= 2)
                def _():
                    write_wait(b)

                assemble(b)
                write_start(gbase + j, b)

                @pl.when(j <= GPW - 3)
                def _():
                    gather_start(j + 2, b)

            return carry

        lax.fori_loop(0, GPW // 2, pair, 0)
        for b in range(2):
            write_wait(b)

    return k(table, idxg)


def kernel(x, embed_weight):
    batch, hist = x.shape
    _, d = embed_weight.shape
    xt = jnp.swapaxes(x, 0, 1).astype(jnp.int32)
    idxg = xt.reshape(NGROUPS, GL)
    out5 = _embed_lookup(embed_weight, idxg)
    return jnp.transpose(out5, (2, 4, 0, 1, 3)).reshape(batch, hist, d)


# speedup vs baseline: 1.3281x; 1.1873x over previous
import functools

import jax
import jax.numpy as jnp
from jax import lax
from jax.experimental import pallas as pl
from jax.experimental.pallas import tpu as pltpu
from jax.experimental.pallas import tpu_sc as plsc

NC = 2
NS = 16
NW = NC * NS
GL = 128
NGROUPS = 1600
GPW = NGROUPS // NW


def _embed_lookup(table, idxg):
    _, D = table.shape
    assert D == 64 and idxg.shape == (NGROUPS, GL)
    mesh = plsc.VectorSubcoreMesh(core_axis_name="c", subcore_axis_name="s")

    @functools.partial(
        pl.kernel,
        mesh=mesh,
        compiler_params=pltpu.CompilerParams(use_tc_tiling_on_sc=False,
                                             needs_layout_passes=False),
        out_type=jax.ShapeDtypeStruct((50, 8, 32, 8, 128), jnp.float32),
        scratch_types=[
            pltpu.VMEM((GPW, GL), jnp.int32),
            pltpu.VMEM((GL, D), jnp.float32),
            pltpu.VMEM((GL, D), jnp.float32),
            pltpu.VMEM((8, 8, GL + 1), jnp.float32),
            pltpu.VMEM((8, 8, GL + 1), jnp.float32),
            pltpu.SemaphoreType.DMA,
            pltpu.SemaphoreType.DMA,
            pltpu.SemaphoreType.DMA,
            pltpu.SemaphoreType.DMA,
        ],
    )
    def k(table_hbm, idx_hbm, out_hbm, idx_all, g0, g1, t0, t1,
          gs0, gs1, ws0, ws1):
        gbuf, tbuf = (g0, g1), (t0, t1)
        gsem, wsem = (gs0, gs1), (ws0, ws1)
        wid = lax.axis_index("s") * NC + lax.axis_index("c")
        gbase = wid * GPW
        pltpu.sync_copy(idx_hbm.at[pl.ds(gbase, GPW)], idx_all)

        iota16 = lax.iota(jnp.int32, 16)
        svec = iota16 & 7
        rvec = [(iota16 + d0) >> 3 for d0 in (0, 16, 32, 48)]

        def gather_start(j, b):
            pltpu.async_copy(table_hbm.at[idx_all.at[j]], gbuf[b], gsem[b])

        def gather_wait(b):
            pltpu.make_async_copy(table_hbm.at[pl.ds(0, GL)], gbuf[b],
                                  gsem[b]).wait()

        def assemble(b):
            for l in range(GL):
                lvec = jnp.full((16,), l, jnp.int32)
                vals = [gbuf[b][l, pl.ds(d0, 16)] for d0 in (0, 16, 32, 48)]
                for i in range(4):
                    plsc.store_scatter(tbuf[b], [rvec[i], svec, lvec],
                                       vals[i])

        def write_start(g, b):
            h = g // 32
            c = g % 32
            pltpu.async_copy(tbuf[b].at[:, :, pl.ds(0, GL)],
                             out_hbm.at[h, :, c], wsem[b])

        def write_wait(b):
            pltpu.make_async_copy(tbuf[b].at[:, :, pl.ds(0, GL)],
                                  out_hbm.at[0, :, 0], wsem[b]).wait()

        for b in range(2):
            gather_start(b, b)

        def pair(i, carry):
            for b in range(2):
                j = 2 * i + b
                gather_wait(b)

                @pl.when(j >= 2)
                def _():
                    write_wait(b)

                assemble(b)
                write_start(gbase + j, b)

                @pl.when(j <= GPW - 3)
                def _():
                    gather_start(j + 2, b)

            return carry

        lax.fori_loop(0, GPW // 2, pair, 0)
        for b in range(2):
            write_wait(b)

    return k(table, idxg)


def kernel(x, embed_weight):
    batch, hist = x.shape
    _, d = embed_weight.shape
    xt = jnp.swapaxes(x, 0, 1).astype(jnp.int32)
    idxg = xt.reshape(NGROUPS, GL)
    out5 = _embed_lookup(embed_weight, idxg)
    return jnp.transpose(out5, (2, 4, 0, 1, 3)).reshape(batch, hist, d)
